# baseline (speedup 1.0000x reference)
import functools

import jax
import jax.numpy as jnp
from jax import lax
from jax.experimental import pallas as pl
from jax.experimental.pallas import tpu as pltpu
from jax.experimental.pallas import tpu_sc as plsc

N = 10000
E = 320000
D = 128
NP = 10240
NC = 2
NS = 16
NW = NC * NS
CH = 128
K = 80
EPAD = NW * K * CH
ROWS_PER_TILE = NP // NS

_mesh = plsc.VectorSubcoreMesh(core_axis_name="c", subcore_axis_name="s")


@functools.partial(
    pl.kernel,
    out_type=jax.ShapeDtypeStruct((NC, NP), jnp.float32),
    mesh=_mesh,
    scratch_types=[
        pltpu.VMEM((K, CH), jnp.int32),
        pltpu.VMEM((CH,), jnp.float32),
        pltpu.VMEM_SHARED((NP,), jnp.float32),
        pltpu.SemaphoreType.DMA,
    ],
)
def _deg_scatter(dst_hbm, zeros1_hbm, out_hbm, dst_v, ones_v, dacc, sem):
    cid = lax.axis_index("c")
    sid = lax.axis_index("s")
    wid = sid * NC + cid
    ez = sid * ROWS_PER_TILE
    pltpu.sync_copy(zeros1_hbm.at[pl.ds(ez, ROWS_PER_TILE)],
                    dacc.at[pl.ds(ez, ROWS_PER_TILE)])
    pltpu.sync_copy(dst_hbm.at[pl.ds(wid * K, K)], dst_v)
    for l in range(CH // 16):
        ones_v[pl.ds(l * 16, 16)] = jnp.ones((16,), jnp.float32)
    plsc.subcore_barrier()

    def body(j, carry):
        pltpu.async_copy(ones_v, dacc.at[dst_v.at[j]], sem, add=True).wait()
        return carry

    lax.fori_loop(0, K, body, 0)
    plsc.subcore_barrier()
    pltpu.sync_copy(dacc.at[pl.ds(ez, ROWS_PER_TILE)],
                    out_hbm.at[cid, pl.ds(ez, ROWS_PER_TILE)])


G = 16

KT = 2 * K
KA = 144
KB = KT - KA


@functools.partial(
    pl.kernel,
    out_type=jax.ShapeDtypeStruct((NC, NP, D), jnp.float32),
    mesh=_mesh,
    scratch_types=[
        pltpu.VMEM((G, CH), jnp.int32),
        pltpu.VMEM((G, CH), jnp.int32),
        pltpu.VMEM((CH, D), jnp.float32),
        pltpu.VMEM((CH, D), jnp.float32),
        pltpu.VMEM_SHARED((NP, D), jnp.float32),
        pltpu.SemaphoreType.DMA,
        pltpu.SemaphoreType.DMA,
        pltpu.SemaphoreType.DMA,
        pltpu.SemaphoreType.DMA,
        pltpu.SemaphoreType.DMA,
    ],
)
def _edge_scatter(src_hbm, dst_hbm, g_hbm, zeros2_hbm, out_hbm,
                  src_v, dst_v, rows0, rows1, acc,
                  sem_g0, sem_g1, sem_s0, sem_s1, sem_z):
    cid = lax.axis_index("c")
    sid = lax.axis_index("s")
    rz = sid * ROWS_PER_TILE
    base_chunk = sid * KT + cid * KA
    ngroups = lax.select(cid == 0, KA // G, KB // G)

    def start_g(j, buf, sem):
        pltpu.async_copy(g_hbm.at[src_v.at[j]], buf, sem)

    def wait_g(buf, sem):
        pltpu.make_async_copy(g_hbm.at[src_v.at[0]], buf, sem).wait()

    def start_s(j, buf, sem):
        pltpu.async_copy(buf, acc.at[dst_v.at[j]], sem, add=True)

    def wait_s(buf, sem):
        pltpu.make_async_copy(buf, acc.at[dst_v.at[0]], sem).wait()

    pltpu.async_copy(zeros2_hbm.at[pl.ds(rz, ROWS_PER_TILE)],
                     acc.at[pl.ds(rz, ROWS_PER_TILE)], sem_z)
    pltpu.make_async_copy(zeros2_hbm.at[pl.ds(rz, ROWS_PER_TILE)],
                          acc.at[pl.ds(rz, ROWS_PER_TILE)], sem_z).wait()
    plsc.subcore_barrier()

    def group(g, carry):
        base = base_chunk + g * G
        pltpu.sync_copy(src_hbm.at[pl.ds(base, G)], src_v)
        pltpu.sync_copy(dst_hbm.at[pl.ds(base, G)], dst_v)
        start_g(0, rows0, sem_g0)

        M = G // 2

        def body(m, c2):
            j0 = 2 * m
            start_g(j0 + 1, rows1, sem_g1)
            wait_g(rows0, sem_g0)
            start_s(j0, rows0, sem_s0)
            wait_s(rows0, sem_s0)

            @pl.when(m + 1 < M)
            def _():
                start_g(j0 + 2, rows0, sem_g0)

            wait_g(rows1, sem_g1)
            start_s(j0 + 1, rows1, sem_s1)
            wait_s(rows1, sem_s1)
            return c2

        lax.fori_loop(0, M, body, 0)
        return carry

    lax.fori_loop(0, ngroups, group, 0)
    plsc.subcore_barrier()
    pltpu.sync_copy(acc.at[pl.ds(rz, ROWS_PER_TILE)],
                    out_hbm.at[cid, pl.ds(rz, ROWS_PER_TILE), :])


def _phase1_body(x_ref, w_ref, d0_ref, d1_ref, g_ref, dinv_ref):
    dinv = lax.rsqrt(d0_ref[...] + d1_ref[...] + 1.0)
    h = jnp.dot(x_ref[...], w_ref[...], preferred_element_type=jnp.float32)
    g_ref[...] = h * dinv
    dinv_ref[...] = dinv


def _phase2_body(s0_ref, s1_ref, g_ref, dinv_ref, b_ref, w_ref, g2_ref):
    dinv = dinv_ref[...]
    x2 = jnp.maximum(
        dinv * (s0_ref[...] + s1_ref[...] + g_ref[...]) + b_ref[...], 0.0)
    g2_ref[...] = jnp.dot(
        x2, w_ref[...], preferred_element_type=jnp.float32) * dinv


def _phase3_body(t0_ref, t1_ref, g2_ref, dinv_ref, b_ref, wfc_ref, bfc_ref,
                 emb_ref, out_ref):
    emb = jnp.maximum(
        dinv_ref[...] * (t0_ref[...] + t1_ref[...] + g2_ref[...])
        + b_ref[...], 0.0)
    emb_ref[...] = emb
    out_ref[...] = jnp.dot(
        emb, wfc_ref[...], preferred_element_type=jnp.float32) + bfc_ref[...]


_phase1 = pl.pallas_call(
    _phase1_body,
    out_shape=(jax.ShapeDtypeStruct((NP, D), jnp.float32),
               jax.ShapeDtypeStruct((NP, 1), jnp.float32)),
)

_phase2 = pl.pallas_call(
    _phase2_body,
    out_shape=jax.ShapeDtypeStruct((NP, D), jnp.float32),
)

_phase3 = pl.pallas_call(
    _phase3_body,
    out_shape=(jax.ShapeDtypeStruct((NP, D), jnp.float32),
               jax.ShapeDtypeStruct((NP, D), jnp.float32)),
)


def kernel(x, edge_index, W1, b1, W2, b2, Wfc, bfc):
    src = edge_index[0]
    dst = edge_index[1]
    pad = EPAD - E
    src_p = jnp.concatenate(
        [src, jnp.zeros((pad,), jnp.int32)]).reshape(NW * K, CH)
    dst_p = jnp.concatenate(
        [dst, jnp.full((pad,), NP - 1, jnp.int32)]).reshape(NW * K, CH)
    xp = jnp.pad(x, ((0, NP - N), (0, 0)))
    zeros1 = jnp.zeros((NP,), jnp.float32)
    zeros2 = jnp.zeros((NP, D), jnp.float32)

    degs = _deg_scatter(dst_p, zeros1)
    d0 = degs[0].reshape(NP, 1)
    d1 = degs[1].reshape(NP, 1)

    g1, dinv = _phase1(xp, W1, d0, d1)
    s = _edge_scatter(src_p, dst_p, g1, zeros2)
    g2 = _phase2(s[0], s[1], g1, dinv, b1.reshape(1, D), W2)
    t = _edge_scatter(src_p, dst_p, g2, zeros2)
    emb_p, out_p = _phase3(t[0], t[1], g2, dinv, b2.reshape(1, D),
                           Wfc, bfc.reshape(1, D))
    return emb_p[:N], out_p[:N]


# speedup vs baseline: 1.1996x; 1.1996x over previous
import functools

import jax
import jax.numpy as jnp
from jax import lax
from jax.experimental import pallas as pl
from jax.experimental.pallas import tpu as pltpu
from jax.experimental.pallas import tpu_sc as plsc

N = 10000
E = 320000
D = 128
NP = 10240
NC = 2
NS = 16
NW = NC * NS
CH = 128
K = 80
EPAD = NW * K * CH
ROWS_PER_TILE = NP // NS

_mesh = plsc.VectorSubcoreMesh(core_axis_name="c", subcore_axis_name="s")


@functools.partial(
    pl.kernel,
    out_type=jax.ShapeDtypeStruct((NC, NP), jnp.float32),
    mesh=_mesh,
    scratch_types=[
        pltpu.VMEM((K, CH), jnp.int32),
        pltpu.VMEM((CH,), jnp.float32),
        pltpu.VMEM_SHARED((NP,), jnp.float32),
        pltpu.SemaphoreType.DMA,
    ],
)
def _deg_scatter(dst_hbm, zeros1_hbm, out_hbm, dst_v, ones_v, dacc, sem):
    cid = lax.axis_index("c")
    sid = lax.axis_index("s")
    wid = sid * NC + cid
    ez = sid * ROWS_PER_TILE
    pltpu.sync_copy(zeros1_hbm.at[pl.ds(ez, ROWS_PER_TILE)],
                    dacc.at[pl.ds(ez, ROWS_PER_TILE)])
    pltpu.sync_copy(dst_hbm.at[pl.ds(wid * K, K)], dst_v)
    for l in range(CH // 16):
        ones_v[pl.ds(l * 16, 16)] = jnp.ones((16,), jnp.float32)
    plsc.subcore_barrier()

    def body(j, carry):
        pltpu.async_copy(ones_v, dacc.at[dst_v.at[j]], sem, add=True).wait()
        return carry

    lax.fori_loop(0, K, body, 0)
    plsc.subcore_barrier()
    pltpu.sync_copy(dacc.at[pl.ds(ez, ROWS_PER_TILE)],
                    out_hbm.at[cid, pl.ds(ez, ROWS_PER_TILE)])


G = 8

KT = 2 * K
KA = 120
KB = KT - KA


@functools.partial(
    pl.kernel,
    out_type=jax.ShapeDtypeStruct((NC, NP, D), jnp.float32),
    mesh=_mesh,
    scratch_types=[
        pltpu.VMEM((G, CH), jnp.int32),
        pltpu.VMEM((G, CH), jnp.int32),
        pltpu.VMEM((CH, D), jnp.float32),
        pltpu.VMEM((CH, D), jnp.float32),
        pltpu.VMEM_SHARED((NP, D), jnp.float32),
        pltpu.SemaphoreType.DMA,
        pltpu.SemaphoreType.DMA,
        pltpu.SemaphoreType.DMA,
        pltpu.SemaphoreType.DMA,
        pltpu.SemaphoreType.DMA,
    ],
)
def _edge_scatter(src_hbm, dst_hbm, g_hbm, zeros2_hbm, out_hbm,
                  src_v, dst_v, rows0, rows1, acc,
                  sem_g0, sem_g1, sem_s0, sem_s1, sem_z):
    cid = lax.axis_index("c")
    sid = lax.axis_index("s")
    rz = sid * ROWS_PER_TILE
    base_chunk = sid * KT + cid * KA
    ngroups = lax.select(cid == 0, KA // G, KB // G)

    def start_g(j, buf, sem):
        pltpu.async_copy(g_hbm.at[src_v.at[j]], buf, sem)

    def wait_g(buf, sem):
        pltpu.make_async_copy(g_hbm.at[src_v.at[0]], buf, sem).wait()

    def start_s(j, buf, sem):
        pltpu.async_copy(buf, acc.at[dst_v.at[j]], sem, add=True)

    def wait_s(buf, sem):
        pltpu.make_async_copy(buf, acc.at[dst_v.at[0]], sem).wait()

    pltpu.async_copy(zeros2_hbm.at[pl.ds(rz, ROWS_PER_TILE)],
                     acc.at[pl.ds(rz, ROWS_PER_TILE)], sem_z)
    pltpu.make_async_copy(zeros2_hbm.at[pl.ds(rz, ROWS_PER_TILE)],
                          acc.at[pl.ds(rz, ROWS_PER_TILE)], sem_z).wait()
    plsc.subcore_barrier()

    def group(g, carry):
        base = base_chunk + g * G
        pltpu.sync_copy(src_hbm.at[pl.ds(base, G)], src_v)
        pltpu.sync_copy(dst_hbm.at[pl.ds(base, G)], dst_v)
        start_g(0, rows0, sem_g0)

        M = G // 2

        def body(m, c2):
            j0 = 2 * m
            start_g(j0 + 1, rows1, sem_g1)
            wait_g(rows0, sem_g0)
            start_s(j0, rows0, sem_s0)
            wait_s(rows0, sem_s0)

            @pl.when(m + 1 < M)
            def _():
                start_g(j0 + 2, rows0, sem_g0)

            wait_g(rows1, sem_g1)
            start_s(j0 + 1, rows1, sem_s1)
            wait_s(rows1, sem_s1)
            return c2

        lax.fori_loop(0, M, body, 0)
        return carry

    lax.fori_loop(0, ngroups, group, 0)
    plsc.subcore_barrier()
    pltpu.sync_copy(acc.at[pl.ds(rz, ROWS_PER_TILE)],
                    out_hbm.at[cid, pl.ds(rz, ROWS_PER_TILE), :])


def _phase1_body(x_ref, w_ref, d0_ref, d1_ref, g_ref, dinv_ref):
    dinv = lax.rsqrt(d0_ref[...] + d1_ref[...] + 1.0)
    h = jnp.dot(x_ref[...], w_ref[...], preferred_element_type=jnp.float32)
    g_ref[...] = h * dinv
    dinv_ref[...] = dinv


def _phase2_body(s0_ref, s1_ref, g_ref, dinv_ref, b_ref, w_ref, g2_ref):
    dinv = dinv_ref[...]
    x2 = jnp.maximum(
        dinv * (s0_ref[...] + s1_ref[...] + g_ref[...]) + b_ref[...], 0.0)
    g2_ref[...] = jnp.dot(
        x2, w_ref[...], preferred_element_type=jnp.float32) * dinv


def _phase3_body(t0_ref, t1_ref, g2_ref, dinv_ref, b_ref, wfc_ref, bfc_ref,
                 emb_ref, out_ref):
    emb = jnp.maximum(
        dinv_ref[...] * (t0_ref[...] + t1_ref[...] + g2_ref[...])
        + b_ref[...], 0.0)
    emb_ref[...] = emb
    out_ref[...] = jnp.dot(
        emb, wfc_ref[...], preferred_element_type=jnp.float32) + bfc_ref[...]


_phase1 = pl.pallas_call(
    _phase1_body,
    out_shape=(jax.ShapeDtypeStruct((NP, D), jnp.float32),
               jax.ShapeDtypeStruct((NP, 1), jnp.float32)),
)

_phase2 = pl.pallas_call(
    _phase2_body,
    out_shape=jax.ShapeDtypeStruct((NP, D), jnp.float32),
)

_phase3 = pl.pallas_call(
    _phase3_body,
    out_shape=(jax.ShapeDtypeStruct((NP, D), jnp.float32),
               jax.ShapeDtypeStruct((NP, D), jnp.float32)),
)


def kernel(x, edge_index, W1, b1, W2, b2, Wfc, bfc):
    src = edge_index[0]
    dst = edge_index[1]
    pad = EPAD - E
    src_p = jnp.concatenate(
        [src, jnp.zeros((pad,), jnp.int32)]).reshape(NW * K, CH)
    dst_p = jnp.concatenate(
        [dst, jnp.full((pad,), NP - 1, jnp.int32)]).reshape(NW * K, CH)
    xp = jnp.pad(x, ((0, NP - N), (0, 0)))
    zeros1 = jnp.zeros((NP,), jnp.float32)
    zeros2 = jnp.zeros((NP, D), jnp.float32)

    degs = _deg_scatter(dst_p, zeros1)
    d0 = degs[0].reshape(NP, 1)
    d1 = degs[1].reshape(NP, 1)

    g1, dinv = _phase1(xp, W1, d0, d1)
    s = _edge_scatter(src_p, dst_p, g1, zeros2)
    g2 = _phase2(s[0], s[1], g1, dinv, b1.reshape(1, D), W2)
    t = _edge_scatter(src_p, dst_p, g2, zeros2)
    emb_p, out_p = _phase3(t[0], t[1], g2, dinv, b2.reshape(1, D),
                           Wfc, bfc.reshape(1, D))
    return emb_p[:N], out_p[:N]
